# baseline (speedup 1.0000x reference)
import functools
import math

import jax
import jax.numpy as jnp
from jax import lax
from jax.experimental import pallas as pl
from jax.experimental.pallas import tpu as pltpu
from jax.experimental.pallas import tpu_sc as plsc

D_MODEL = 64
SCALE = math.sqrt(D_MODEL)

NUM_CORES = 2
NUM_SUBCORES = 16
NUM_WORKERS = NUM_CORES * NUM_SUBCORES

B_TOTAL = 4096 * 50
ROWS_PER_WORKER = B_TOTAL // NUM_WORKERS
CHUNK = 640
NUM_CHUNKS = ROWS_PER_WORKER // CHUNK
SUBGATHER = 128
NUM_SUB = CHUNK // SUBGATHER


def _emb_kernel(lut_hbm, idx_hbm, out_hbm, idx_v, rows_v, sem):
    wid = lax.axis_index("s") * NUM_CORES + lax.axis_index("c")
    base = wid * ROWS_PER_WORKER

    def chunk_body(c, _):
        row0 = base + c * CHUNK
        pltpu.sync_copy(idx_hbm.at[pl.ds(row0, CHUNK)], idx_v)
        copies = []
        for j in range(NUM_SUB):
            copies.append(pltpu.async_copy(
                lut_hbm.at[idx_v.at[pl.ds(j * SUBGATHER, SUBGATHER)]],
                rows_v.at[pl.ds(j * SUBGATHER, SUBGATHER)],
                sem,
            ))
        for cp in copies:
            cp.wait()

        def mul_body(r, _):
            for k in range(D_MODEL // 16):
                sl = pl.ds(k * 16, 16)
                rows_v[r, sl] = rows_v[r, sl] * SCALE
            return None
        lax.fori_loop(0, CHUNK, mul_body, None)

        pltpu.sync_copy(rows_v, out_hbm.at[pl.ds(row0, CHUNK)])
        return None

    lax.fori_loop(0, NUM_CHUNKS, chunk_body, None)


@jax.jit
def kernel(x, lut):
    idx = x.reshape(-1).astype(jnp.int32)
    mesh = plsc.VectorSubcoreMesh(core_axis_name="c", subcore_axis_name="s")
    out = pl.kernel(
        _emb_kernel,
        mesh=mesh,
        compiler_params=pltpu.CompilerParams(use_tc_tiling_on_sc=False),
        out_type=jax.ShapeDtypeStruct((B_TOTAL, D_MODEL), jnp.float32),
        scratch_types=[
            pltpu.VMEM((CHUNK,), jnp.int32),
            pltpu.VMEM((CHUNK, D_MODEL), jnp.float32),
            pltpu.SemaphoreType.DMA,
        ],
    )(lut, idx)
    return out.reshape(x.shape[0], x.shape[1], D_MODEL)


# speedup vs baseline: 1.0428x; 1.0428x over previous
import math

import jax
import jax.numpy as jnp
from jax import lax
from jax.experimental import pallas as pl
from jax.experimental.pallas import tpu as pltpu
from jax.experimental.pallas import tpu_sc as plsc

D_MODEL = 64
SCALE = math.sqrt(D_MODEL)

NUM_CORES = 2
NUM_SUBCORES = 16
NUM_WORKERS = NUM_CORES * NUM_SUBCORES

B_TOTAL = 4096 * 50
ROWS_PER_WORKER = B_TOTAL // NUM_WORKERS
CHUNK = 640
NUM_CHUNKS = ROWS_PER_WORKER // CHUNK
SUBGATHER = 128
NUM_SUB = CHUNK // SUBGATHER
ROW_UNROLL = 4


def _emb_kernel(lut_hbm, idx_hbm, out_hbm,
                idx0, idx1, rows0, rows1, gsem0, gsem1, wsem0, wsem1):
    wid = lax.axis_index("s") * NUM_CORES + lax.axis_index("c")
    base = wid * ROWS_PER_WORKER
    idx_bufs = (idx0, idx1)
    row_bufs = (rows0, rows1)
    gsems = (gsem0, gsem1)
    wsems = (wsem0, wsem1)

    def fire(c):
        b = c % 2
        row0 = base + c * CHUNK
        pltpu.sync_copy(idx_hbm.at[pl.ds(row0, CHUNK)], idx_bufs[b])
        cps = []
        for j in range(NUM_SUB):
            cps.append(pltpu.async_copy(
                lut_hbm.at[idx_bufs[b].at[pl.ds(j * SUBGATHER, SUBGATHER)]],
                row_bufs[b].at[pl.ds(j * SUBGATHER, SUBGATHER)],
                gsems[b],
            ))
        return cps

    def write(c):
        b = c % 2
        row0 = base + c * CHUNK
        return pltpu.async_copy(row_bufs[b], out_hbm.at[pl.ds(row0, CHUNK)],
                                wsems[b])

    pending_g = {0: fire(0)}
    pending_w = {}
    for c in range(NUM_CHUNKS):
        b = c % 2
        if c + 1 < NUM_CHUNKS:
            if c - 1 in pending_w:
                pending_w.pop(c - 1).wait()
            pending_g[c + 1] = fire(c + 1)
        for cp in pending_g.pop(c):
            cp.wait()

        rows = row_bufs[b]

        def mul_body(i, _):
            for rr in range(ROW_UNROLL):
                r = i * ROW_UNROLL + rr
                for k in range(D_MODEL // 16):
                    sl = pl.ds(k * 16, 16)
                    rows[r, sl] = rows[r, sl] * SCALE
            return None
        lax.fori_loop(0, CHUNK // ROW_UNROLL, mul_body, None)

        pending_w[c] = write(c)
    for cp in pending_w.values():
        cp.wait()


@jax.jit
def kernel(x, lut):
    idx = x.reshape(-1).astype(jnp.int32)
    mesh = plsc.VectorSubcoreMesh(core_axis_name="c", subcore_axis_name="s")
    out = pl.kernel(
        _emb_kernel,
        mesh=mesh,
        compiler_params=pltpu.CompilerParams(use_tc_tiling_on_sc=False),
        out_type=jax.ShapeDtypeStruct((B_TOTAL, D_MODEL), jnp.float32),
        scratch_types=[
            pltpu.VMEM((CHUNK,), jnp.int32),
            pltpu.VMEM((CHUNK,), jnp.int32),
            pltpu.VMEM((CHUNK, D_MODEL), jnp.float32),
            pltpu.VMEM((CHUNK, D_MODEL), jnp.float32),
            pltpu.SemaphoreType.DMA,
            pltpu.SemaphoreType.DMA,
            pltpu.SemaphoreType.DMA,
            pltpu.SemaphoreType.DMA,
        ],
    )(lut, idx)
    return out.reshape(x.shape[0], x.shape[1], D_MODEL)


# speedup vs baseline: 1.0497x; 1.0066x over previous
import math

import jax
import jax.numpy as jnp
from jax import lax
from jax.experimental import pallas as pl
from jax.experimental.pallas import tpu as pltpu
from jax.experimental.pallas import tpu_sc as plsc

D_MODEL = 64
SCALE = math.sqrt(D_MODEL)

NUM_CORES = 2
NUM_SUBCORES = 16
NUM_WORKERS = NUM_CORES * NUM_SUBCORES

B_TOTAL = 4096 * 50
ROWS_PER_WORKER = B_TOTAL // NUM_WORKERS
CHUNK = 256
NUM_CHUNKS = ROWS_PER_WORKER // CHUNK
SUBGATHER = 128
NUM_SUB = CHUNK // SUBGATHER
ROW_UNROLL = 4
D_PAD = 128


def _emb_kernel(lut_hbm, idx_hbm, out_hbm,
                idx0, idx1, rows0, rows1, gsem0, gsem1, wsem0, wsem1):
    wid = lax.axis_index("s") * NUM_CORES + lax.axis_index("c")
    base = wid * ROWS_PER_WORKER
    idx_bufs = (idx0, idx1)
    row_bufs = (rows0, rows1)
    gsems = (gsem0, gsem1)
    wsems = (wsem0, wsem1)

    def fire(c):
        b = c % 2
        row0 = base + c * CHUNK
        pltpu.sync_copy(idx_hbm.at[pl.ds(row0, CHUNK)], idx_bufs[b])
        cps = []
        for j in range(NUM_SUB):
            cps.append(pltpu.async_copy(
                lut_hbm.at[idx_bufs[b].at[pl.ds(j * SUBGATHER, SUBGATHER)]],
                row_bufs[b].at[pl.ds(j * SUBGATHER, SUBGATHER)],
                gsems[b],
            ))
        return cps

    def write(c):
        b = c % 2
        row0 = base + c * CHUNK
        return pltpu.async_copy(row_bufs[b], out_hbm.at[pl.ds(row0, CHUNK)],
                                wsems[b])

    pending_g = {0: fire(0)}
    pending_w = {}
    for c in range(NUM_CHUNKS):
        b = c % 2
        if c + 1 < NUM_CHUNKS:
            if c - 1 in pending_w:
                pending_w.pop(c - 1).wait()
            pending_g[c + 1] = fire(c + 1)
        for cp in pending_g.pop(c):
            cp.wait()

        rows = row_bufs[b]

        def mul_body(i, _):
            for rr in range(ROW_UNROLL):
                r = i * ROW_UNROLL + rr
                for k in range(D_MODEL // 16):
                    sl = pl.ds(k * 16, 16)
                    rows[r, sl] = rows[r, sl] * SCALE
            return None
        lax.fori_loop(0, CHUNK // ROW_UNROLL, mul_body, None)

        pending_w[c] = write(c)
    for cp in pending_w.values():
        cp.wait()


@jax.jit
def kernel(x, lut):
    idx = x.reshape(-1).astype(jnp.int32)
    lutp = jnp.pad(lut, ((0, 0), (0, D_PAD - D_MODEL)))
    mesh = plsc.VectorSubcoreMesh(core_axis_name="c", subcore_axis_name="s")
    out = pl.kernel(
        _emb_kernel,
        mesh=mesh,
        compiler_params=pltpu.CompilerParams(use_tc_tiling_on_sc=True),
        out_type=jax.ShapeDtypeStruct((B_TOTAL, D_PAD), jnp.float32),
        scratch_types=[
            pltpu.VMEM((CHUNK,), jnp.int32),
            pltpu.VMEM((CHUNK,), jnp.int32),
            pltpu.VMEM((CHUNK, D_PAD), jnp.float32),
            pltpu.VMEM((CHUNK, D_PAD), jnp.float32),
            pltpu.SemaphoreType.DMA,
            pltpu.SemaphoreType.DMA,
            pltpu.SemaphoreType.DMA,
            pltpu.SemaphoreType.DMA,
        ],
    )(lutp, idx)
    return out[:, :D_MODEL].reshape(x.shape[0], x.shape[1], D_MODEL)
